# baseline (speedup 1.0000x reference)
import functools

import jax
import jax.numpy as jnp
from jax import lax
from jax.experimental import pallas as pl
from jax.experimental.pallas import tpu as pltpu
from jax.experimental.pallas import tpu_sc as plsc

_SEQ = 200
_BATCH = 4096
_EMB = 64
_VOCAB = 1000000
_NC = 2
_NS = 16
_NW = _NC * _NS
_BPW = _BATCH // _NW

_BCOL = 20480
_NBLK = (_VOCAB + _BCOL - 1) // _BCOL
_T2PAD = _NBLK * _BCOL


def _project(tableT, W1, b1, W2, b2):
    def body(tb_ref, w1_ref, w2_ref, b1_ref, b2_ref, p_ref, c_ref):
        mt = lax.dot_general(
            w2_ref[...], w1_ref[...], (((1,), (0,)), ((), ())),
            preferred_element_type=jnp.float32,
        )
        rt = lax.dot_general(
            mt, tb_ref[...], (((1,), (0,)), ((), ())),
            preferred_element_type=jnp.float32,
            precision=lax.Precision.HIGHEST,
        )

        def rne16(v):
            u = lax.bitcast_convert_type(v, jnp.uint32)
            return (u + jnp.uint32(0x7FFF) + ((u >> 16) & jnp.uint32(1))) >> 16

        packed = (rne16(rt[0, :]) << 16) | rne16(rt[1, :])
        p_ref[...] = lax.bitcast_convert_type(packed, jnp.float32)
        c_ref[...] = lax.dot_general(
            w2_ref[...], b1_ref[...], (((1,), (0,)), ((), ())),
            preferred_element_type=jnp.float32,
        ) + b2_ref[...]

    return pl.pallas_call(
        body,
        grid=(_NBLK,),
        in_specs=[
            pl.BlockSpec((_EMB, _BCOL), lambda i: (0, i)),
            pl.BlockSpec((128, _EMB), lambda i: (0, 0)),
            pl.BlockSpec((2, 128), lambda i: (0, 0)),
            pl.BlockSpec((128, 16), lambda i: (0, 0)),
            pl.BlockSpec((2, 16), lambda i: (0, 0)),
        ],
        out_specs=[
            pl.BlockSpec((_BCOL,), lambda i: (i,)),
            pl.BlockSpec((2, 16), lambda i: (0, 0)),
        ],
        out_shape=[
            jax.ShapeDtypeStruct((_T2PAD,), jnp.float32),
            jax.ShapeDtypeStruct((2, 16), jnp.float32),
        ],
    )(tableT, W1, W2,
      jnp.broadcast_to(b1.reshape(-1, 1), (128, 16)),
      jnp.broadcast_to(b2.reshape(-1, 1), (2, 16)))


def _sc_pool(x, t2p, c):
    mesh = plsc.VectorSubcoreMesh(core_axis_name="c", subcore_axis_name="s")
    hi_mask = jnp.uint32(0xFFFF0000)

    @functools.partial(
        pl.kernel,
        mesh=mesh,
        out_type=jax.ShapeDtypeStruct((2, _BATCH), jnp.float32),
        scratch_types=[
            pltpu.VMEM((_SEQ, _BPW), jnp.int32),
            pltpu.VMEM((4, _BPW), jnp.float32),
            pltpu.VMEM((_BPW,), jnp.float32),
            pltpu.VMEM((_BPW,), jnp.float32),
            pltpu.VMEM((2, 16), jnp.float32),
            pltpu.SemaphoreType.DMA,
            pltpu.SemaphoreType.DMA,
            pltpu.SemaphoreType.DMA,
            pltpu.SemaphoreType.DMA,
        ],
        compiler_params=pltpu.CompilerParams(
            use_tc_tiling_on_sc=False, needs_layout_passes=False),
    )
    def body(x_hbm, p_hbm, c_hbm, out_hbm, idx_v, vp_v,
             ra_v, rb_v, c_v, s0, s1, s2, s3):
        sems = (s0, s1, s2, s3)
        wid = lax.axis_index("s") * _NC + lax.axis_index("c")
        base = wid * _BPW
        pltpu.sync_copy(x_hbm.at[:, pl.ds(base, _BPW)], idx_v)
        pltpu.sync_copy(c_hbm, c_v)

        for d in range(4):
            pltpu.async_copy(p_hbm.at[idx_v.at[d]], vp_v.at[d], sems[d])

        nch = _BPW // 16
        zeros = [jnp.zeros((16,), jnp.float32) for _ in range(2 * nch)]

        def step(i, acc):
            acc = list(acc)
            for d in range(4):
                s = 4 * i + d
                pltpu.make_async_copy(
                    p_hbm.at[idx_v.at[s]], vp_v.at[d], sems[d]).wait()
                for j in range(nch):
                    u = plsc.bitcast(vp_v[d, pl.ds(16 * j, 16)], jnp.uint32)
                    av = plsc.bitcast(u & hi_mask, jnp.float32)
                    bv = plsc.bitcast(u << 16, jnp.float32)
                    acc[j] = acc[j] + av
                    acc[nch + j] = acc[nch + j] + bv

                @pl.when(s + 4 < _SEQ)
                def _():
                    pltpu.async_copy(
                        p_hbm.at[idx_v.at[s + 4]], vp_v.at[d], sems[d])

            return tuple(acc)

        acc = lax.fori_loop(0, _SEQ // 4, step, tuple(zeros))

        inv = jnp.float32(1.0 / _SEQ)
        ca = c_v[0]
        cb = c_v[1]
        for j in range(nch):
            ra_v[pl.ds(16 * j, 16)] = acc[j] * inv + ca
            rb_v[pl.ds(16 * j, 16)] = acc[nch + j] * inv + cb
        pltpu.sync_copy(ra_v, out_hbm.at[0, pl.ds(base, _BPW)])
        pltpu.sync_copy(rb_v, out_hbm.at[1, pl.ds(base, _BPW)])

    return body(x, t2p, c)


def kernel(x, table, W1, b1, W2, b2):
    x = x.astype(jnp.int32)
    t2p, c = _project(table.T, W1, b1, W2, b2)
    out = _sc_pool(x, t2p, c)
    return out.T


# speedup vs baseline: 1.2522x; 1.2522x over previous
import functools

import jax
import jax.numpy as jnp
from jax import lax
from jax.experimental import pallas as pl
from jax.experimental.pallas import tpu as pltpu
from jax.experimental.pallas import tpu_sc as plsc

_SEQ = 200
_BATCH = 4096
_EMB = 64
_VOCAB = 1000000
_NC = 2
_NS = 16
_NW = _NC * _NS
_BPW = _BATCH // _NW

_BCOL = 20480
_NBLK = (_VOCAB + _BCOL - 1) // _BCOL
_T2PAD = _NBLK * _BCOL


def _project(tableT, W1, b1, W2, b2):
    def body(tb_ref, w1_ref, w2_ref, b1_ref, b2_ref, p_ref, c_ref):
        mt = lax.dot_general(
            w2_ref[...], w1_ref[...], (((1,), (0,)), ((), ())),
            preferred_element_type=jnp.float32,
        )
        rt = lax.dot_general(
            mt, tb_ref[...], (((1,), (0,)), ((), ())),
            preferred_element_type=jnp.float32,
        )

        def rne16(v):
            u = lax.bitcast_convert_type(v, jnp.uint32)
            return (u + jnp.uint32(0x7FFF) + ((u >> 16) & jnp.uint32(1))) >> 16

        packed = (rne16(rt[0, :]) << 16) | rne16(rt[1, :])
        p_ref[...] = lax.bitcast_convert_type(packed, jnp.float32)
        c_ref[...] = lax.dot_general(
            w2_ref[...], b1_ref[...], (((1,), (0,)), ((), ())),
            preferred_element_type=jnp.float32,
        ) + b2_ref[...]

    return pl.pallas_call(
        body,
        grid=(_NBLK,),
        in_specs=[
            pl.BlockSpec((_EMB, _BCOL), lambda i: (0, i)),
            pl.BlockSpec((128, _EMB), lambda i: (0, 0)),
            pl.BlockSpec((2, 128), lambda i: (0, 0)),
            pl.BlockSpec((128, 16), lambda i: (0, 0)),
            pl.BlockSpec((2, 16), lambda i: (0, 0)),
        ],
        out_specs=[
            pl.BlockSpec((_BCOL,), lambda i: (i,)),
            pl.BlockSpec((2, 16), lambda i: (0, 0)),
        ],
        out_shape=[
            jax.ShapeDtypeStruct((_T2PAD,), jnp.float32),
            jax.ShapeDtypeStruct((2, 16), jnp.float32),
        ],
    )(tableT, W1, W2,
      jnp.broadcast_to(b1.reshape(-1, 1), (128, 16)),
      jnp.broadcast_to(b2.reshape(-1, 1), (2, 16)))


def _sc_pool(x, t2p, c):
    mesh = plsc.VectorSubcoreMesh(core_axis_name="c", subcore_axis_name="s")
    hi_mask = jnp.uint32(0xFFFF0000)

    @functools.partial(
        pl.kernel,
        mesh=mesh,
        out_type=jax.ShapeDtypeStruct((2, _BATCH), jnp.float32),
        scratch_types=[
            pltpu.VMEM((_SEQ, _BPW), jnp.int32),
            pltpu.VMEM((4, _BPW), jnp.float32),
            pltpu.VMEM((_BPW,), jnp.float32),
            pltpu.VMEM((_BPW,), jnp.float32),
            pltpu.VMEM((2, 16), jnp.float32),
            pltpu.SemaphoreType.DMA,
            pltpu.SemaphoreType.DMA,
            pltpu.SemaphoreType.DMA,
            pltpu.SemaphoreType.DMA,
        ],
        compiler_params=pltpu.CompilerParams(
            use_tc_tiling_on_sc=False, needs_layout_passes=False),
    )
    def body(x_hbm, p_hbm, c_hbm, out_hbm, idx_v, vp_v,
             ra_v, rb_v, c_v, s0, s1, s2, s3):
        sems = (s0, s1, s2, s3)
        wid = lax.axis_index("s") * _NC + lax.axis_index("c")
        base = wid * _BPW
        pltpu.sync_copy(x_hbm.at[:, pl.ds(base, _BPW)], idx_v)
        pltpu.sync_copy(c_hbm, c_v)

        for d in range(4):
            pltpu.async_copy(p_hbm.at[idx_v.at[d]], vp_v.at[d], sems[d])

        nch = _BPW // 16
        zeros = [jnp.zeros((16,), jnp.float32) for _ in range(2 * nch)]

        def step(i, acc):
            acc = list(acc)
            for d in range(4):
                s = 4 * i + d
                pltpu.make_async_copy(
                    p_hbm.at[idx_v.at[s]], vp_v.at[d], sems[d]).wait()
                for j in range(nch):
                    u = plsc.bitcast(vp_v[d, pl.ds(16 * j, 16)], jnp.uint32)
                    av = plsc.bitcast(u & hi_mask, jnp.float32)
                    bv = plsc.bitcast(u << 16, jnp.float32)
                    acc[j] = acc[j] + av
                    acc[nch + j] = acc[nch + j] + bv

                @pl.when(s + 4 < _SEQ)
                def _():
                    pltpu.async_copy(
                        p_hbm.at[idx_v.at[s + 4]], vp_v.at[d], sems[d])

            return tuple(acc)

        acc = lax.fori_loop(0, _SEQ // 4, step, tuple(zeros))

        inv = jnp.float32(1.0 / _SEQ)
        ca = c_v[0]
        cb = c_v[1]
        for j in range(nch):
            ra_v[pl.ds(16 * j, 16)] = acc[j] * inv + ca
            rb_v[pl.ds(16 * j, 16)] = acc[nch + j] * inv + cb
        pltpu.sync_copy(ra_v, out_hbm.at[0, pl.ds(base, _BPW)])
        pltpu.sync_copy(rb_v, out_hbm.at[1, pl.ds(base, _BPW)])

    return body(x, t2p, c)


def kernel(x, table, W1, b1, W2, b2):
    x = x.astype(jnp.int32)
    t2p, c = _project(table.T, W1, b1, W2, b2)
    out = _sc_pool(x, t2p, c)
    return out.T
